# baseline (speedup 1.0000x reference)
import jax
import jax.numpy as jnp
from jax import lax
from jax.experimental import pallas as pl
from jax.experimental.pallas import tpu as pltpu
from jax.experimental.pallas import tpu_sc as plsc

DIMS = (119, 4, 12, 12, 10, 6, 6, 2, 2)
BASES = (0, 119, 123, 135, 147, 157, 163, 169, 171)
TOTAL_ROWS = 173
D = 128
N = 100000
NC = 9
NW = 32
BLOCKS = N // 16
BPW = BLOCKS // NW
EXTRA = BLOCKS - BPW * NW
CHUNK_BLOCKS = 5
CHUNK = CHUNK_BLOCKS * 16
NCHUNKS = BPW // CHUNK_BLOCKS


def _body(x_hbm, emb0, emb1, emb2, emb3, emb4, emb5, emb6, emb7, emb8,
          out_hbm, table_v, x_v, out_v):
    embs = (emb0, emb1, emb2, emb3, emb4, emb5, emb6, emb7, emb8)
    for i in range(9):
        pltpu.sync_copy(embs[i], table_v.at[pl.ds(BASES[i] * D, DIMS[i] * D)])

    wid = lax.axis_index("s") * 2 + lax.axis_index("c")
    start_block = wid * BPW + jnp.minimum(wid, EXTRA)

    lanes = lax.iota(jnp.int32, 16)

    def compute_block(b):
        rows = lanes + b * 16
        addrs = []
        for i in range(NC):
            xi = plsc.load_gather(x_v, [rows * NC + i])
            addrs.append((xi + BASES[i]) * D)
        out_base = rows * D

        def dbody(d0, carry):
            for dd in range(4):
                d = d0 * 4 + dd
                acc = plsc.load_gather(table_v, [addrs[0] + d])
                for i in range(1, NC):
                    acc = acc + plsc.load_gather(table_v, [addrs[i] + d])
                plsc.store_scatter(out_v, [out_base + d], acc)
            return carry

        lax.fori_loop(0, D // 4, dbody, 0)

    def chunk_body(c, carry):
        row0 = (start_block + c * CHUNK_BLOCKS) * 16
        pltpu.sync_copy(x_hbm.at[pl.ds(row0 * NC, CHUNK * NC)], x_v)
        for b in range(CHUNK_BLOCKS):
            compute_block(b)
        pltpu.sync_copy(out_v.at[pl.ds(0, CHUNK * D)],
                        out_hbm.at[pl.ds(row0 * D, CHUNK * D)])
        return carry

    lax.fori_loop(0, NCHUNKS, chunk_body, 0)

    @pl.when(wid < EXTRA)
    def _():
        row0 = (start_block + BPW) * 16
        pltpu.sync_copy(x_hbm.at[pl.ds(row0 * NC, 16 * NC)],
                        x_v.at[pl.ds(0, 16 * NC)])
        compute_block(0)
        pltpu.sync_copy(out_v.at[pl.ds(0, 16 * D)],
                        out_hbm.at[pl.ds(row0 * D, 16 * D)])


@jax.jit
def kernel(x, emb0, emb1, emb2, emb3, emb4, emb5, emb6, emb7, emb8):
    mesh = plsc.VectorSubcoreMesh(core_axis_name="c", subcore_axis_name="s")
    run = pl.kernel(
        _body,
        out_type=jax.ShapeDtypeStruct((N * D,), jnp.float32),
        mesh=mesh,
        compiler_params=pltpu.CompilerParams(needs_layout_passes=False),
        scratch_types=[
            pltpu.VMEM((TOTAL_ROWS * D,), jnp.float32),
            pltpu.VMEM((CHUNK * NC,), jnp.int32),
            pltpu.VMEM((CHUNK * D,), jnp.float32),
        ],
    )
    flat = run(x.reshape(-1), *(e.reshape(-1) for e in
                                (emb0, emb1, emb2, emb3, emb4, emb5, emb6,
                                 emb7, emb8)))
    return flat.reshape(N, D)


# speedup vs baseline: 2.8573x; 2.8573x over previous
import jax
import jax.numpy as jnp
from jax import lax
from jax.experimental import pallas as pl
from jax.experimental.pallas import tpu as pltpu
from jax.experimental.pallas import tpu_sc as plsc

DIMS = (119, 4, 12, 12, 10, 6, 6, 2, 2)
D = 128
N = 100000
NC = 9
NW = 32
BLOCKS = N // 16
BPW = BLOCKS // NW
EXTRA = BLOCKS - BPW * NW
CHUNK_BLOCKS = 5
CHUNK = CHUNK_BLOCKS * 16
NCHUNKS = BPW // CHUNK_BLOCKS

RAW_BASES = (0, 4, 16, 28, 38, 44, 50, 52)
RAW_ROWS = 54
G0, G1, G2, G3 = 0, 119, 215, 335
TABLE_ROWS = 407


def _body(x_hbm, emb0, emb1, emb2, emb3, emb4, emb5, emb6, emb7, emb8,
          out_hbm, raw_v, table_v, x_v, out_v):
    pltpu.sync_copy(emb0, table_v.at[pl.ds(G0 * D, 119 * D)])
    small = (emb1, emb2, emb3, emb4, emb5, emb6, emb7, emb8)
    for i, e in enumerate(small):
        pltpu.sync_copy(e, raw_v.at[pl.ds(RAW_BASES[i] * D, DIMS[i + 1] * D)])

    def build3(base_out, ra, na, rb, nb, rc, nc):
        def row(r, carry):
            a = r // nb
            b = r % nb
            for k in range(D // 16):
                va = raw_v[pl.ds((ra + a) * D + k * 16, 16)]
                vb = raw_v[pl.ds((rb + b) * D + k * 16, 16)]
                vab = va + vb
                for c in range(nc):
                    vc = raw_v[pl.ds((rc + c) * D + k * 16, 16)]
                    table_v[pl.ds((base_out + r * nc + c) * D + k * 16, 16)] = (
                        vab + vc)
            return carry
        lax.fori_loop(0, na * nb, row, 0)

    def build2(base_out, ra, na, rb, nb):
        def row(r, carry):
            a = r // nb
            b = r % nb
            for k in range(D // 16):
                va = raw_v[pl.ds((ra + a) * D + k * 16, 16)]
                vb = raw_v[pl.ds((rb + b) * D + k * 16, 16)]
                table_v[pl.ds((base_out + r) * D + k * 16, 16)] = va + vb
            return carry
        lax.fori_loop(0, na * nb, row, 0)

    build3(G1, RAW_BASES[0], 4, RAW_BASES[1], 12, RAW_BASES[7], 2)
    build2(G2, RAW_BASES[2], 12, RAW_BASES[3], 10)
    build3(G3, RAW_BASES[4], 6, RAW_BASES[5], 6, RAW_BASES[6], 2)

    wid = lax.axis_index("s") * 2 + lax.axis_index("c")
    start_block = wid * BPW + jnp.minimum(wid, EXTRA)
    lanes = lax.iota(jnp.int32, 16)

    def compute_block(b):
        rows = lanes + b * 16
        xs = [plsc.load_gather(x_v, [rows * NC + i]) for i in range(NC)]
        a0 = (xs[0] + G0) * D
        a1 = (xs[1] * 24 + xs[2] * 2 + xs[8] + G1) * D
        a2 = (xs[3] * 10 + xs[4] + G2) * D
        a3 = (xs[5] * 12 + xs[6] * 2 + xs[7] + G3) * D
        ob = rows * D

        @plsc.parallel_loop(0, D, unroll=8)
        def _(d):
            acc = plsc.load_gather(table_v, [a0 + d])
            acc = acc + plsc.load_gather(table_v, [a1 + d])
            acc = acc + plsc.load_gather(table_v, [a2 + d])
            acc = acc + plsc.load_gather(table_v, [a3 + d])
            plsc.store_scatter(out_v, [ob + d], acc)

    def chunk_body(c, carry):
        row0 = (start_block + c * CHUNK_BLOCKS) * 16
        pltpu.sync_copy(x_hbm.at[pl.ds(row0 * NC, CHUNK * NC)], x_v)
        for b in range(CHUNK_BLOCKS):
            compute_block(b)
        pltpu.sync_copy(out_v.at[pl.ds(0, CHUNK * D)],
                        out_hbm.at[pl.ds(row0 * D, CHUNK * D)])
        return carry

    lax.fori_loop(0, NCHUNKS, chunk_body, 0)

    @pl.when(wid < EXTRA)
    def _():
        row0 = (start_block + BPW) * 16
        pltpu.sync_copy(x_hbm.at[pl.ds(row0 * NC, 16 * NC)],
                        x_v.at[pl.ds(0, 16 * NC)])
        compute_block(0)
        pltpu.sync_copy(out_v.at[pl.ds(0, 16 * D)],
                        out_hbm.at[pl.ds(row0 * D, 16 * D)])


@jax.jit
def kernel(x, emb0, emb1, emb2, emb3, emb4, emb5, emb6, emb7, emb8):
    mesh = plsc.VectorSubcoreMesh(core_axis_name="c", subcore_axis_name="s")
    run = pl.kernel(
        _body,
        out_type=jax.ShapeDtypeStruct((N * D,), jnp.float32),
        mesh=mesh,
        compiler_params=pltpu.CompilerParams(needs_layout_passes=False),
        scratch_types=[
            pltpu.VMEM((RAW_ROWS * D,), jnp.float32),
            pltpu.VMEM((TABLE_ROWS * D,), jnp.float32),
            pltpu.VMEM((CHUNK * NC,), jnp.int32),
            pltpu.VMEM((CHUNK * D,), jnp.float32),
        ],
    )
    flat = run(x.reshape(-1), *(e.reshape(-1) for e in
                                (emb0, emb1, emb2, emb3, emb4, emb5, emb6,
                                 emb7, emb8)))
    return flat.reshape(N, D)


# speedup vs baseline: 11.7171x; 4.1007x over previous
import jax
import jax.numpy as jnp
from jax import lax
from jax.experimental import pallas as pl
from jax.experimental.pallas import tpu as pltpu
from jax.experimental.pallas import tpu_sc as plsc

DIMS = (119, 4, 12, 12, 10, 6, 6, 2, 2)
D = 128
N = 100000
NC = 9
NW = 32
BLOCKS = N // 16
BPW = BLOCKS // NW
EXTRA = BLOCKS - BPW * NW
CHUNK_BLOCKS = 5
CHUNK = CHUNK_BLOCKS * 16
NCHUNKS = BPW // CHUNK_BLOCKS

RAW_BASES = (0, 4, 16, 28, 38, 44, 50, 52)
RAW_ROWS = 54
G0, G1, G2, G3 = 0, 119, 215, 335
TABLE_ROWS = 407


def _body(x_hbm, emb0, emb1, emb2, emb3, emb4, emb5, emb6, emb7, emb8,
          out_hbm, raw_v, table_v, x_v, out_v):
    pltpu.sync_copy(emb0, table_v.at[pl.ds(G0 * D, 119 * D)])
    small = (emb1, emb2, emb3, emb4, emb5, emb6, emb7, emb8)
    for i, e in enumerate(small):
        pltpu.sync_copy(e, raw_v.at[pl.ds(RAW_BASES[i] * D, DIMS[i + 1] * D)])

    def build3(base_out, ra, na, rb, nb, rc, nc):
        def row(r, carry):
            a = r // nb
            b = r % nb
            for k in range(D // 16):
                va = raw_v[pl.ds((ra + a) * D + k * 16, 16)]
                vb = raw_v[pl.ds((rb + b) * D + k * 16, 16)]
                vab = va + vb
                for c in range(nc):
                    vc = raw_v[pl.ds((rc + c) * D + k * 16, 16)]
                    table_v[pl.ds((base_out + r * nc + c) * D + k * 16, 16)] = (
                        vab + vc)
            return carry
        lax.fori_loop(0, na * nb, row, 0)

    def build2(base_out, ra, na, rb, nb):
        def row(r, carry):
            a = r // nb
            b = r % nb
            for k in range(D // 16):
                va = raw_v[pl.ds((ra + a) * D + k * 16, 16)]
                vb = raw_v[pl.ds((rb + b) * D + k * 16, 16)]
                table_v[pl.ds((base_out + r) * D + k * 16, 16)] = va + vb
            return carry
        lax.fori_loop(0, na * nb, row, 0)

    build3(G1, RAW_BASES[0], 4, RAW_BASES[1], 12, RAW_BASES[7], 2)
    build2(G2, RAW_BASES[2], 12, RAW_BASES[3], 10)
    build3(G3, RAW_BASES[4], 6, RAW_BASES[5], 6, RAW_BASES[6], 2)

    wid = lax.axis_index("s") * 2 + lax.axis_index("c")
    start_block = wid * BPW + jnp.minimum(wid, EXTRA)

    def compute_sample(s):
        xvec = x_v[pl.ds(s * NC, 16)]
        xs = [xvec[i] for i in range(NC)]
        b0 = (xs[0] + G0) * D
        b1 = (xs[1] * 24 + xs[2] * 2 + xs[8] + G1) * D
        b2 = (xs[3] * 10 + xs[4] + G2) * D
        b3 = (xs[5] * 12 + xs[6] * 2 + xs[7] + G3) * D
        so = s * D
        for dc in range(D // 16):
            o = dc * 16
            acc = table_v[pl.ds(b0 + o, 16)] + table_v[pl.ds(b1 + o, 16)]
            acc = acc + table_v[pl.ds(b2 + o, 16)]
            acc = acc + table_v[pl.ds(b3 + o, 16)]
            out_v[pl.ds(so + o, 16)] = acc

    def chunk_body(c, carry):
        row0 = (start_block + c * CHUNK_BLOCKS) * 16
        pltpu.sync_copy(x_hbm.at[pl.ds(row0 * NC, CHUNK * NC)],
                        x_v.at[pl.ds(0, CHUNK * NC)])

        @plsc.parallel_loop(0, CHUNK, unroll=2)
        def _(s):
            compute_sample(s)

        pltpu.sync_copy(out_v.at[pl.ds(0, CHUNK * D)],
                        out_hbm.at[pl.ds(row0 * D, CHUNK * D)])
        return carry

    lax.fori_loop(0, NCHUNKS, chunk_body, 0)

    @pl.when(wid < EXTRA)
    def _():
        row0 = (start_block + BPW) * 16
        pltpu.sync_copy(x_hbm.at[pl.ds(row0 * NC, 16 * NC)],
                        x_v.at[pl.ds(0, 16 * NC)])

        @plsc.parallel_loop(0, 16, unroll=2)
        def _(s):
            compute_sample(s)

        pltpu.sync_copy(out_v.at[pl.ds(0, 16 * D)],
                        out_hbm.at[pl.ds(row0 * D, 16 * D)])


@jax.jit
def kernel(x, emb0, emb1, emb2, emb3, emb4, emb5, emb6, emb7, emb8):
    mesh = plsc.VectorSubcoreMesh(core_axis_name="c", subcore_axis_name="s")
    run = pl.kernel(
        _body,
        out_type=jax.ShapeDtypeStruct((N * D,), jnp.float32),
        mesh=mesh,
        compiler_params=pltpu.CompilerParams(needs_layout_passes=False),
        scratch_types=[
            pltpu.VMEM((RAW_ROWS * D,), jnp.float32),
            pltpu.VMEM((TABLE_ROWS * D,), jnp.float32),
            pltpu.VMEM((CHUNK * NC + 16,), jnp.int32),
            pltpu.VMEM((CHUNK * D,), jnp.float32),
        ],
    )
    flat = run(x.reshape(-1), *(e.reshape(-1) for e in
                                (emb0, emb1, emb2, emb3, emb4, emb5, emb6,
                                 emb7, emb8)))
    return flat.reshape(N, D)


# speedup vs baseline: 14.4180x; 1.2305x over previous
import jax
import jax.numpy as jnp
from jax import lax
from jax.experimental import pallas as pl
from jax.experimental.pallas import tpu as pltpu
from jax.experimental.pallas import tpu_sc as plsc

DIMS = (119, 4, 12, 12, 10, 6, 6, 2, 2)
D = 128
N = 100000
NC = 9
NW = 32
CHUNK = 120
NCHUNKS = 26
SPW = CHUNK * NCHUNKS
EXTRA = (N - SPW * NW) // 16

RAW_BASES = (0, 4, 16, 28, 38, 44, 50, 52)
RAW_ROWS = 54
G0, G1, G2, G3 = 0, 119, 215, 335
TABLE_ROWS = 407


def _body(x_hbm, emb0, emb1, emb2, emb3, emb4, emb5, emb6, emb7, emb8,
          out_hbm, raw_v, table_v, x_v0, x_v1, out_v0, out_v1,
          sem_x0, sem_x1, sem_o0, sem_o1):
    x_bufs = (x_v0, x_v1)
    out_bufs = (out_v0, out_v1)
    sem_x = (sem_x0, sem_x1)
    sem_o = (sem_o0, sem_o1)

    pltpu.sync_copy(emb0, table_v.at[pl.ds(G0 * D, 119 * D)])
    small = (emb1, emb2, emb3, emb4, emb5, emb6, emb7, emb8)
    for i, e in enumerate(small):
        pltpu.sync_copy(e, raw_v.at[pl.ds(RAW_BASES[i] * D, DIMS[i + 1] * D)])

    def build3(base_out, ra, na, rb, nb, rc, nc):
        def row(r, carry):
            a = r // nb
            b = r % nb
            for k in range(D // 16):
                va = raw_v[pl.ds((ra + a) * D + k * 16, 16)]
                vb = raw_v[pl.ds((rb + b) * D + k * 16, 16)]
                vab = va + vb
                for c in range(nc):
                    vc = raw_v[pl.ds((rc + c) * D + k * 16, 16)]
                    table_v[pl.ds((base_out + r * nc + c) * D + k * 16, 16)] = (
                        vab + vc)
            return carry
        lax.fori_loop(0, na * nb, row, 0)

    def build2(base_out, ra, na, rb, nb):
        def row(r, carry):
            a = r // nb
            b = r % nb
            for k in range(D // 16):
                va = raw_v[pl.ds((ra + a) * D + k * 16, 16)]
                vb = raw_v[pl.ds((rb + b) * D + k * 16, 16)]
                table_v[pl.ds((base_out + r) * D + k * 16, 16)] = va + vb
            return carry
        lax.fori_loop(0, na * nb, row, 0)

    build3(G1, RAW_BASES[0], 4, RAW_BASES[1], 12, RAW_BASES[7], 2)
    build2(G2, RAW_BASES[2], 12, RAW_BASES[3], 10)
    build3(G3, RAW_BASES[4], 6, RAW_BASES[5], 6, RAW_BASES[6], 2)

    wid = lax.axis_index("s") * 2 + lax.axis_index("c")
    start_sample = wid * SPW + jnp.minimum(wid, EXTRA) * 16

    def compute_sample(s, x_v, out_v):
        xvec = x_v[pl.ds(s * NC, 16)]
        xs = [xvec[i] for i in range(NC)]
        b0 = (xs[0] + G0) * D
        b1 = (xs[1] * 24 + xs[2] * 2 + xs[8] + G1) * D
        b2 = (xs[3] * 10 + xs[4] + G2) * D
        b3 = (xs[5] * 12 + xs[6] * 2 + xs[7] + G3) * D
        so = s * D
        for dc in range(D // 16):
            o = dc * 16
            acc = table_v[pl.ds(b0 + o, 16)] + table_v[pl.ds(b1 + o, 16)]
            acc = acc + table_v[pl.ds(b2 + o, 16)]
            acc = acc + table_v[pl.ds(b3 + o, 16)]
            out_v[pl.ds(so + o, 16)] = acc

    def start_x(c, b):
        row0 = start_sample + c * CHUNK
        pltpu.async_copy(x_hbm.at[pl.ds(row0 * NC, CHUNK * NC)],
                         x_bufs[b].at[pl.ds(0, CHUNK * NC)], sem_x[b])

    def wait_x(b):
        pltpu.make_async_copy(x_hbm.at[pl.ds(0, CHUNK * NC)],
                              x_bufs[b].at[pl.ds(0, CHUNK * NC)],
                              sem_x[b]).wait()

    def start_o(c, b):
        row0 = start_sample + c * CHUNK
        pltpu.async_copy(out_bufs[b],
                         out_hbm.at[pl.ds(row0 * D, CHUNK * D)], sem_o[b])

    def wait_o(b):
        pltpu.make_async_copy(out_bufs[b],
                              out_hbm.at[pl.ds(0, CHUNK * D)],
                              sem_o[b]).wait()

    start_x(0, 0)
    start_x(1, 1)

    def ring_body(i, carry):
        for b in range(2):
            c = i * 2 + b
            wait_x(b)

            @pl.when(c >= 2)
            def _():
                wait_o(b)

            @plsc.parallel_loop(0, CHUNK, unroll=4)
            def _(s):
                compute_sample(s, x_bufs[b], out_bufs[b])

            start_o(c, b)

            @pl.when(c + 2 < NCHUNKS)
            def _():
                start_x(c + 2, b)
        return carry

    lax.fori_loop(0, NCHUNKS // 2, ring_body, 0)
    wait_o(0)
    wait_o(1)

    @pl.when(wid < EXTRA)
    def _():
        row0 = start_sample + SPW
        pltpu.sync_copy(x_hbm.at[pl.ds(row0 * NC, 16 * NC)],
                        x_v0.at[pl.ds(0, 16 * NC)])

        @plsc.parallel_loop(0, 16, unroll=4)
        def _(s):
            compute_sample(s, x_v0, out_v0)

        pltpu.sync_copy(out_v0.at[pl.ds(0, 16 * D)],
                        out_hbm.at[pl.ds(row0 * D, 16 * D)])


@jax.jit
def kernel(x, emb0, emb1, emb2, emb3, emb4, emb5, emb6, emb7, emb8):
    mesh = plsc.VectorSubcoreMesh(core_axis_name="c", subcore_axis_name="s")
    run = pl.kernel(
        _body,
        out_type=jax.ShapeDtypeStruct((N * D,), jnp.float32),
        mesh=mesh,
        compiler_params=pltpu.CompilerParams(needs_layout_passes=False),
        scratch_types=[
            pltpu.VMEM((RAW_ROWS * D,), jnp.float32),
            pltpu.VMEM((TABLE_ROWS * D,), jnp.float32),
            pltpu.VMEM((CHUNK * NC + 16,), jnp.int32),
            pltpu.VMEM((CHUNK * NC + 16,), jnp.int32),
            pltpu.VMEM((CHUNK * D,), jnp.float32),
            pltpu.VMEM((CHUNK * D,), jnp.float32),
            pltpu.SemaphoreType.DMA,
            pltpu.SemaphoreType.DMA,
            pltpu.SemaphoreType.DMA,
            pltpu.SemaphoreType.DMA,
        ],
    )
    flat = run(x.reshape(-1), *(e.reshape(-1) for e in
                                (emb0, emb1, emb2, emb3, emb4, emb5, emb6,
                                 emb7, emb8)))
    return flat.reshape(N, D)
